# Initial kernel scaffold; baseline (speedup 1.0000x reference)
import jax
import jax.numpy as jnp
from jax.experimental import pallas as pl


def kernel(context, weight):
    raise NotImplementedError("write your pallas kernel here")


# speedup vs baseline: 7.9806x; 7.9806x over previous
import functools
import math

import jax
import jax.numpy as jnp
from jax.experimental import pallas as pl
from jax.experimental.pallas import tpu as pltpu

N, D, P, K = 4096, 512, 16, 128
PD = P * D

BM = 512
BK = 512
KBLKS = PD // BK
BN_X = 256


def _build_x_kernel(c_ref, w_ref, hi_ref):
    c = c_ref[...]
    w = w_ref[...]
    cf = c[:, None, :] * w[None, :, :]
    norm = jnp.sqrt(jnp.sum(cf * cf, axis=2, keepdims=True))
    x = cf / jnp.maximum(norm, 1e-12)
    hi_ref[...] = x.astype(jnp.bfloat16)


def _attn_topk_kernel(lhs_ref, rhs_ref, out_ref, skey_ref):
    k = pl.program_id(1)
    dims = (((1,), (1,)), ((), ()))
    part = jax.lax.dot_general(lhs_ref[...], rhs_ref[...], dims,
                               preferred_element_type=jnp.float32)

    @pl.when(k == 0)
    def _():
        out_ref[...] = part

    @pl.when(k > 0)
    def _():
        out_ref[...] = out_ref[...] + part

    @pl.when(k == KBLKS - 1)
    def _():
        att = out_ref[...] * jnp.float32(1.0 / P)
        bits = jax.lax.bitcast_convert_type(att, jnp.int32)
        skey = bits ^ ((bits >> 31) & jnp.int32(0x7FFFFFFF))
        skey_ref[...] = skey

        t0 = jnp.full((BM, 1), jnp.int32(-2147483648), dtype=jnp.int32)

        def body(i, t):
            cand = t + (jnp.int32(1) << (jnp.int32(31) - i))
            cnt = jnp.sum((skey_ref[...] >= cand).astype(jnp.int32),
                          axis=1, keepdims=True)
            return jnp.where(cnt >= K, cand, t)

        t = jax.lax.fori_loop(0, 32, body, t0)
        keep = skey_ref[...] >= t
        out_ref[...] = jnp.where(keep, att, jnp.float32(0.0))


@jax.jit
def kernel(context, weight):
    x3 = pl.pallas_call(
        _build_x_kernel,
        grid=(N // BN_X,),
        in_specs=[
            pl.BlockSpec((BN_X, D), lambda i: (i, 0)),
            pl.BlockSpec((P, D), lambda i: (0, 0)),
        ],
        out_specs=pl.BlockSpec((BN_X, P, D), lambda i: (i, 0, 0)),
        out_shape=jax.ShapeDtypeStruct((N, P, D), jnp.bfloat16),
    )(context, weight)
    x = x3.reshape(N, PD)

    out = pl.pallas_call(
        _attn_topk_kernel,
        grid=(N // BM, KBLKS),
        in_specs=[
            pl.BlockSpec((BM, BK), lambda m, k: (m, k)),
            pl.BlockSpec((N, BK), lambda m, k: (0, k)),
        ],
        out_specs=pl.BlockSpec((BM, N), lambda m, k: (m, 0)),
        out_shape=jax.ShapeDtypeStruct((N, N), jnp.float32),
        scratch_shapes=[pltpu.VMEM((BM, N), jnp.int32)],
        compiler_params=pltpu.CompilerParams(
            dimension_semantics=("parallel", "arbitrary")),
    )(x, x)
    return out
